# baseline (speedup 1.0000x reference)
import functools

import jax
import jax.numpy as jnp
import numpy as np
from jax import lax
from jax.experimental import pallas as pl

_BINS = 6
_WIDTH = 256.0 / _BINS
_BIG = 3.0e38


def _nrm(val, tmn, tmx, lo, hi):
    den = tmx - tmn
    scaled = (val - tmn) / jnp.where(den == 0.0, 1.0, den) * (hi - lo) + lo
    return jnp.where(den == 0.0, jnp.zeros_like(scaled), scaled)


def _stage1_body(x_ref, sum_ref, border_ref):
    step = pl.program_id(0)
    x = x_ref[...]
    K, H, W = x.shape
    HW = float(H * W)
    rows = lax.broadcasted_iota(jnp.int32, (H, W), 0)
    cols = lax.broadcasted_iota(jnp.int32, (H, W), 1)
    interior = (rows > 0) & (rows < H - 1) & (cols > 0) & (cols < W - 1)
    xb = jnp.where(interior[None, :, :], x, 0.0)
    mn = jnp.min(xb, axis=(1, 2))
    mx = jnp.max(xb, axis=(1, 2))
    den = mx - mn

    thr1 = [mn + den * (float(j) / 6.0) for j in (1, 2, 3, 4, 5)]
    thr2 = [mn + den * (float(k + 1) / 1536.0) for k in (1, 2, 3, 4, 5)]
    cge1 = [jnp.sum(jnp.where(xb >= thr1[j][:, None, None], 1.0, 0.0),
                    axis=(1, 2)) for j in range(5)]
    cge2 = [jnp.sum(jnp.where(xb >= thr2[k][:, None, None], 1.0, 0.0),
                    axis=(1, 2)) for k in range(5)]
    c1 = [HW - cge1[0]] + [cge1[k] - cge1[k + 1] for k in range(4)] + [cge1[4]]
    c2 = [HW - cge2[0]] + [cge2[k] - cge2[k + 1] for k in range(4)] + [cge2[4]]

    v = [-jnp.log(c1[j] / HW + 0.0001) for j in range(6)]
    pres = [c2[j] > 0.0 for j in range(6)]
    pmn = functools.reduce(jnp.minimum, [jnp.where(pres[j], v[j], _BIG) for j in range(6)])
    pmx = functools.reduce(jnp.maximum, [jnp.where(pres[j], v[j], -_BIG) for j in range(6)])
    nd = [_nrm(v[j], pmn, pmx, 0.0, 1.0) for j in range(6)]
    meand = sum(c2[j] * nd[j] for j in range(6)) / HW
    dmn = functools.reduce(jnp.minimum, [jnp.where(pres[j], nd[j], _BIG) for j in range(6)])
    dmx = functools.reduce(jnp.maximum, [jnp.where(pres[j], nd[j], -_BIG) for j in range(6)])
    w1 = (dmx - meand) ** 2
    rt = [_nrm(nd[j], dmn, dmx, 0.0, 1.0) * w1 for j in range(6)]

    nb = float(2 * H + 2 * W - 4)
    lane = lax.iota(jnp.int32, K)
    chan0 = (lane == 0) & (step == 0)
    nbv = jnp.where(chan0, 0.0, nb)
    ge = [jnp.where(thr2[k] <= 0.0, 1.0, 0.0) for k in range(5)]
    e = [1.0 - ge[0]] + [ge[k] - ge[k + 1] for k in range(4)] + [ge[4]]
    cc = [c2[j] - nbv * e[j] for j in range(6)]
    presr = [cc[j] > 0.0 for j in range(6)]
    rmn = functools.reduce(jnp.minimum, [jnp.where(presr[j], rt[j], _BIG) for j in range(6)])
    rmx = functools.reduce(jnp.maximum, [jnp.where(presr[j], rt[j], -_BIG) for j in range(6)])
    rmn = jnp.where(nbv > 0.0, jnp.minimum(rmn, 0.0), rmn)
    rmx = jnp.where(nbv > 0.0, jnp.maximum(rmx, 0.0), rmx)
    meanr = sum(cc[j] * rt[j] for j in range(6)) / HW
    w2 = (rmx - meanr) ** 2
    T = [_nrm(rt[j], rmn, rmx, 0.0, 1.0) * w2 for j in range(6)]

    val = jnp.broadcast_to(T[0][:, None, None], xb.shape)
    for k in range(5):
        val = jnp.where(xb >= thr2[k][:, None, None], T[k + 1][:, None, None], val)
    contrib = jnp.sum(val, axis=0)

    bval = T[0]
    for k in range(5):
        bval = jnp.where(thr2[k] <= 0.0, T[k + 1], bval)

    @pl.when(step == 0)
    def _():
        sum_ref[...] = contrib
        border_ref[...] = bval[0:1][None]

    @pl.when(step != 0)
    def _():
        sum_ref[...] += contrib


def _stage1(chw, kblk):
    C, H, W = chw.shape
    assert C % kblk == 0
    return pl.pallas_call(
        _stage1_body,
        grid=(C // kblk,),
        in_specs=[pl.BlockSpec((kblk, H, W), lambda i: (i, 0, 0))],
        out_specs=[
            pl.BlockSpec((H, W), lambda i: (0, 0)),
            pl.BlockSpec((1, 1), lambda i: (0, 0)),
        ],
        out_shape=[
            jax.ShapeDtypeStruct((H, W), jnp.float32),
            jax.ShapeDtypeStruct((1, 1), jnp.float32),
        ],
    )(chw)


def _stage2_body(m0, b0, m1, b1, m2, b2, m3, b3,
                 a256, a256t, a128, a128t, a64, a64t,
                 sum_out, groups_out):
    def layer_nr(m_ref, b_ref, a_ref, at_ref):
        m = m_ref[...]
        H, W = m.shape
        rows = lax.broadcasted_iota(jnp.int32, (H, W), 0)
        cols = lax.broadcasted_iota(jnp.int32, (H, W), 1)
        interior = (rows > 0) & (rows < H - 1) & (cols > 0) & (cols < W - 1)
        b = b_ref[...]
        m = jnp.where(interior, m, b[0, 0])
        p = _nrm(m, jnp.min(m), jnp.max(m), 0.0, 1.0)
        r1 = jnp.dot(p, at_ref[...], precision=lax.Precision.HIGHEST)
        r = jnp.dot(a_ref[...], r1, precision=lax.Precision.HIGHEST)
        w = (jnp.max(r) - jnp.mean(r)) ** 2
        return _nrm(r, jnp.min(r), jnp.max(r), 0.0, 1.0) * w

    n0 = layer_nr(m0, b0, a256, a256t)
    n1 = layer_nr(m1, b1, a256, a256t)
    n2 = layer_nr(m2, b2, a128, a128t)
    n3 = layer_nr(m3, b3, a64, a64t)
    cols = [n0 + n1, n2, n3]
    total = jnp.zeros((240, 240), jnp.float32)
    for k, c in enumerate(cols):
        cn = _nrm(c, jnp.min(c), jnp.max(c), 0.0, 256.0)
        groups_out[k] = cn
        total = total + cn
    sum_out[...] = total


@functools.cache
def _resize_mats(n, out=240):
    f32 = np.float32
    inv_scale = f32(n) / f32(out)
    kernel_scale = np.maximum(inv_scale, f32(1.0))
    sample_f = (np.arange(out, dtype=f32) + f32(0.5)) * inv_scale - f32(0.5)
    x = np.abs(sample_f[None, :] - np.arange(n, dtype=f32)[:, None]) / kernel_scale
    w = np.maximum(f32(0.0), f32(1.0) - x).astype(f32)
    tot = w.sum(axis=0, keepdims=True, dtype=f32)
    w = np.where(np.abs(tot) > f32(1e-8), (w / tot).astype(f32), f32(0.0))
    ok = (sample_f >= -0.5) & (sample_f <= n - 0.5)
    w = np.where(ok[None, :], w, f32(0.0)).astype(f32)
    a = np.ascontiguousarray(w.T)
    return a, np.ascontiguousarray(w)


def _stage2(m0, b0, m1, b1, m2, b2, m3, b3):
    a256, a256t = _resize_mats(m0.shape[0])
    a128, a128t = _resize_mats(m2.shape[0])
    a64, a64t = _resize_mats(m3.shape[0])
    return pl.pallas_call(
        _stage2_body,
        out_shape=[
            jax.ShapeDtypeStruct((240, 240), jnp.float32),
            jax.ShapeDtypeStruct((3, 240, 240), jnp.float32),
        ],
    )(m0, b0, m1, b1, m2, b2, m3, b3,
      jnp.asarray(a256), jnp.asarray(a256t),
      jnp.asarray(a128), jnp.asarray(a128t),
      jnp.asarray(a64), jnp.asarray(a64t))


def _kblk(c, cap):
    k = 1
    for d in range(1, min(c, cap) + 1):
        if c % d == 0:
            k = d
    return k


def kernel(layer0, layer1, layer2, layer3):
    m0, b0 = _stage1(layer0[0], _kblk(layer0.shape[1], 32))
    m1, b1 = _stage1(layer1[0], _kblk(layer1.shape[1], 32))
    m2, b2 = _stage1(layer2[0], _kblk(layer2.shape[1], 64))
    m3, b3 = _stage1(layer3[0], _kblk(layer3.shape[1], 128))
    s, g = _stage2(m0, b0, m1, b1, m2, b2, m3, b3)
    return s, jnp.transpose(g, (1, 2, 0))


# speedup vs baseline: 1.0413x; 1.0413x over previous
import functools

import jax
import jax.numpy as jnp
import numpy as np
from jax import lax
from jax.experimental import pallas as pl

_BINS = 6
_WIDTH = 256.0 / _BINS
_BIG = 3.0e38


def _nrm(val, tmn, tmx, lo, hi):
    den = tmx - tmn
    scaled = (val - tmn) / jnp.where(den == 0.0, 1.0, den) * (hi - lo) + lo
    return jnp.where(den == 0.0, jnp.zeros_like(scaled), scaled)


def _stage1_body(x_ref, sum_ref, border_ref):
    step = pl.program_id(0)
    x = x_ref[...]
    K, H, W = x.shape
    HW = float(H * W)
    rows = lax.broadcasted_iota(jnp.int32, (H, W), 0)
    cols = lax.broadcasted_iota(jnp.int32, (H, W), 1)
    interior = (rows > 0) & (rows < H - 1) & (cols > 0) & (cols < W - 1)
    xb = jnp.where(interior[None, :, :], x, 0.0)
    mn = jnp.min(xb, axis=(1, 2))
    mx = jnp.max(xb, axis=(1, 2))
    den = mx - mn

    thr1 = [mn + den * (float(j) / 6.0) for j in (1, 2, 3, 4, 5)]
    thr2 = [mn + den * (float(k + 1) / 1536.0) for k in (1, 2, 3, 4, 5)]
    cge1 = [jnp.sum(jnp.where(xb >= thr1[j][:, None, None], 1.0, 0.0),
                    axis=(1, 2)) for j in range(5)]
    cge2 = [jnp.sum(jnp.where(xb >= thr2[k][:, None, None], 1.0, 0.0),
                    axis=(1, 2)) for k in range(5)]
    c1 = [HW - cge1[0]] + [cge1[k] - cge1[k + 1] for k in range(4)] + [cge1[4]]
    c2 = [HW - cge2[0]] + [cge2[k] - cge2[k + 1] for k in range(4)] + [cge2[4]]

    v = [-jnp.log(c1[j] / HW + 0.0001) for j in range(6)]
    pres = [c2[j] > 0.0 for j in range(6)]
    pmn = functools.reduce(jnp.minimum, [jnp.where(pres[j], v[j], _BIG) for j in range(6)])
    pmx = functools.reduce(jnp.maximum, [jnp.where(pres[j], v[j], -_BIG) for j in range(6)])
    nd = [_nrm(v[j], pmn, pmx, 0.0, 1.0) for j in range(6)]
    meand = sum(c2[j] * nd[j] for j in range(6)) / HW
    dmn = functools.reduce(jnp.minimum, [jnp.where(pres[j], nd[j], _BIG) for j in range(6)])
    dmx = functools.reduce(jnp.maximum, [jnp.where(pres[j], nd[j], -_BIG) for j in range(6)])
    w1 = (dmx - meand) ** 2
    rt = [_nrm(nd[j], dmn, dmx, 0.0, 1.0) * w1 for j in range(6)]

    nb = float(2 * H + 2 * W - 4)
    lane = lax.iota(jnp.int32, K)
    chan0 = (lane == 0) & (step == 0)
    nbv = jnp.where(chan0, 0.0, nb)
    ge = [jnp.where(thr2[k] <= 0.0, 1.0, 0.0) for k in range(5)]
    e = [1.0 - ge[0]] + [ge[k] - ge[k + 1] for k in range(4)] + [ge[4]]
    cc = [c2[j] - nbv * e[j] for j in range(6)]
    presr = [cc[j] > 0.0 for j in range(6)]
    rmn = functools.reduce(jnp.minimum, [jnp.where(presr[j], rt[j], _BIG) for j in range(6)])
    rmx = functools.reduce(jnp.maximum, [jnp.where(presr[j], rt[j], -_BIG) for j in range(6)])
    rmn = jnp.where(nbv > 0.0, jnp.minimum(rmn, 0.0), rmn)
    rmx = jnp.where(nbv > 0.0, jnp.maximum(rmx, 0.0), rmx)
    meanr = sum(cc[j] * rt[j] for j in range(6)) / HW
    w2 = (rmx - meanr) ** 2
    T = [_nrm(rt[j], rmn, rmx, 0.0, 1.0) * w2 for j in range(6)]

    val = jnp.broadcast_to(T[0][:, None, None], xb.shape)
    for k in range(5):
        val = jnp.where(xb >= thr2[k][:, None, None], T[k + 1][:, None, None], val)
    contrib = jnp.sum(val, axis=0)

    bval = T[0]
    for k in range(5):
        bval = jnp.where(thr2[k] <= 0.0, T[k + 1], bval)

    @pl.when(step == 0)
    def _():
        sum_ref[...] = contrib
        border_ref[...] = bval[0:1][None]

    @pl.when(step != 0)
    def _():
        sum_ref[...] += contrib


def _stage1(chw, kblk):
    C, H, W = chw.shape
    assert C % kblk == 0
    return pl.pallas_call(
        _stage1_body,
        grid=(C // kblk,),
        in_specs=[pl.BlockSpec((kblk, H, W), lambda i: (i, 0, 0))],
        out_specs=[
            pl.BlockSpec((H, W), lambda i: (0, 0)),
            pl.BlockSpec((1, 1), lambda i: (0, 0)),
        ],
        out_shape=[
            jax.ShapeDtypeStruct((H, W), jnp.float32),
            jax.ShapeDtypeStruct((1, 1), jnp.float32),
        ],
    )(chw)


def _stage2_body(m0, b0, m1, b1, m2, b2, m3, b3,
                 a256, a256t, a128, a128t, a64, a64t,
                 sum_out, groups_out):
    def layer_nr(m_ref, b_ref, a_ref, at_ref):
        m = m_ref[...]
        H, W = m.shape
        rows = lax.broadcasted_iota(jnp.int32, (H, W), 0)
        cols = lax.broadcasted_iota(jnp.int32, (H, W), 1)
        interior = (rows > 0) & (rows < H - 1) & (cols > 0) & (cols < W - 1)
        b = b_ref[...]
        m = jnp.where(interior, m, b[0, 0])
        p = _nrm(m, jnp.min(m), jnp.max(m), 0.0, 1.0)
        r1 = jnp.dot(p, at_ref[...], precision=lax.Precision.HIGHEST)
        r = jnp.dot(a_ref[...], r1, precision=lax.Precision.HIGHEST)
        w = (jnp.max(r) - jnp.mean(r)) ** 2
        return _nrm(r, jnp.min(r), jnp.max(r), 0.0, 1.0) * w

    n0 = layer_nr(m0, b0, a256, a256t)
    n1 = layer_nr(m1, b1, a256, a256t)
    n2 = layer_nr(m2, b2, a128, a128t)
    n3 = layer_nr(m3, b3, a64, a64t)
    cols = [n0 + n1, n2, n3]
    total = jnp.zeros((240, 240), jnp.float32)
    for k, c in enumerate(cols):
        cn = _nrm(c, jnp.min(c), jnp.max(c), 0.0, 256.0)
        groups_out[k] = cn
        total = total + cn
    sum_out[...] = total


@functools.cache
def _resize_mats(n, out=240):
    f32 = np.float32
    inv_scale = f32(n) / f32(out)
    kernel_scale = np.maximum(inv_scale, f32(1.0))
    sample_f = (np.arange(out, dtype=f32) + f32(0.5)) * inv_scale - f32(0.5)
    x = np.abs(sample_f[None, :] - np.arange(n, dtype=f32)[:, None]) / kernel_scale
    w = np.maximum(f32(0.0), f32(1.0) - x).astype(f32)
    tot = w.sum(axis=0, keepdims=True, dtype=f32)
    w = np.where(np.abs(tot) > f32(1e-8), (w / tot).astype(f32), f32(0.0))
    ok = (sample_f >= -0.5) & (sample_f <= n - 0.5)
    w = np.where(ok[None, :], w, f32(0.0)).astype(f32)
    a = np.ascontiguousarray(w.T)
    return a, np.ascontiguousarray(w)


def _stage2(m0, b0, m1, b1, m2, b2, m3, b3):
    a256, a256t = _resize_mats(m0.shape[0])
    a128, a128t = _resize_mats(m2.shape[0])
    a64, a64t = _resize_mats(m3.shape[0])
    return pl.pallas_call(
        _stage2_body,
        out_shape=[
            jax.ShapeDtypeStruct((240, 240), jnp.float32),
            jax.ShapeDtypeStruct((3, 240, 240), jnp.float32),
        ],
    )(m0, b0, m1, b1, m2, b2, m3, b3,
      jnp.asarray(a256), jnp.asarray(a256t),
      jnp.asarray(a128), jnp.asarray(a128t),
      jnp.asarray(a64), jnp.asarray(a64t))


def _kblk(c, cap):
    k = 1
    for d in range(1, min(c, cap) + 1):
        if c % d == 0:
            k = d
    return k


def kernel(layer0, layer1, layer2, layer3):
    m0, b0 = _stage1(layer0[0], _kblk(layer0.shape[1], 16))
    m1, b1 = _stage1(layer1[0], _kblk(layer1.shape[1], 16))
    m2, b2 = _stage1(layer2[0], _kblk(layer2.shape[1], 32))
    m3, b3 = _stage1(layer3[0], _kblk(layer3.shape[1], 128))
    s, g = _stage2(m0, b0, m1, b1, m2, b2, m3, b3)
    return s, jnp.transpose(g, (1, 2, 0))
